# Initial kernel scaffold; baseline (speedup 1.0000x reference)
import jax
import jax.numpy as jnp
from jax.experimental import pallas as pl


def kernel(x, pe):
    raise NotImplementedError("write your pallas kernel here")


# speedup vs baseline: 5.0458x; 5.0458x over previous
import functools

import jax
import jax.numpy as jnp
from jax import lax
from jax.experimental import pallas as pl
from jax.experimental.pallas import tpu as pltpu
from jax.experimental.pallas import tpu_sc as plsc

_NC = 2
_NS = 16
_NW = _NC * _NS
_L = 128
_KCH = 8


@functools.lru_cache(maxsize=None)
def _build(V, D, B):
    assert B % (_NW * _L) == 0
    b_per_w = B // _NW
    k_total = b_per_w // _L
    assert k_total % _KCH == 0
    n_ch = k_total // _KCH
    rows_per_ch = _KCH * _L

    mesh = plsc.VectorSubcoreMesh(core_axis_name="c", subcore_axis_name="s")

    @functools.partial(
        pl.kernel,
        mesh=mesh,
        out_type=jax.ShapeDtypeStruct((B, D), jnp.float32),
        compiler_params=pltpu.CompilerParams(use_tc_tiling_on_sc=False),
        scratch_types=[
            pltpu.VMEM((_KCH, _L), jnp.int32),
            pltpu.VMEM((rows_per_ch, D), jnp.float32),
            pltpu.SemaphoreType.DMA,
        ],
    )
    def gather_kernel(idx_hbm, table_hbm, out_hbm, idx_v, rows_v, sem):
        wid = lax.axis_index("s") * _NC + lax.axis_index("c")

        def body(ch, carry):
            pltpu.sync_copy(idx_hbm.at[wid, pl.ds(ch * _KCH, _KCH)], idx_v)
            copies = [
                pltpu.async_copy(
                    table_hbm.at[idx_v.at[j]],
                    rows_v.at[pl.ds(j * _L, _L)],
                    sem,
                )
                for j in range(_KCH)
            ]
            for c in copies:
                c.wait()
            base = wid * b_per_w + ch * rows_per_ch
            pltpu.sync_copy(rows_v, out_hbm.at[pl.ds(base, rows_per_ch)])
            return carry

        lax.fori_loop(0, n_ch, body, 0)

    return gather_kernel


def kernel(x, pe):
    V, D = pe.shape
    B = x.size
    xf = x.reshape(-1).astype(jnp.int32)
    idx3 = xf.reshape(_NW, B // (_NW * _L), _L)
    out = _build(V, D, B)(idx3, pe)
    return out.reshape(*x.shape, D)


# speedup vs baseline: 5.3108x; 1.0525x over previous
import functools

import jax
import jax.numpy as jnp
from jax import lax
from jax.experimental import pallas as pl
from jax.experimental.pallas import tpu as pltpu
from jax.experimental.pallas import tpu_sc as plsc

_NC = 2
_NS = 16
_NW = _NC * _NS
_L = 128
_KCH = 8
_RCH = _KCH * _L


@functools.lru_cache(maxsize=None)
def _build(V, D, B):
    assert B % (_NW * _RCH) == 0
    b_per_w = B // _NW
    n_ch = b_per_w // _RCH

    mesh = plsc.VectorSubcoreMesh(core_axis_name="c", subcore_axis_name="s")

    @functools.partial(
        pl.kernel,
        mesh=mesh,
        out_type=jax.ShapeDtypeStruct((B, D), jnp.float32),
        compiler_params=pltpu.CompilerParams(use_tc_tiling_on_sc=False),
        scratch_types=[
            pltpu.VMEM((2, _KCH, _L), jnp.int32),
            pltpu.VMEM((2, _RCH, D), jnp.float32),
            pltpu.SemaphoreType.DMA((2,)),
            pltpu.SemaphoreType.DMA((2,)),
            pltpu.SemaphoreType.DMA((2,)),
        ],
    )
    def gather_kernel(idx_hbm, table_hbm, out_hbm, idx_v, rows_v, isem, gsem, osem):
        wid = lax.axis_index("s") * _NC + lax.axis_index("c")
        base = wid * b_per_w

        def idx_copy(ch, buf):
            return pltpu.make_async_copy(
                idx_hbm.at[wid, pl.ds(ch * _KCH, _KCH)], idx_v.at[buf], isem.at[buf]
            )

        def out_copy(ch, buf):
            return pltpu.make_async_copy(
                rows_v.at[buf], out_hbm.at[pl.ds(base + ch * _RCH, _RCH)], osem.at[buf]
            )

        def gather_drain(buf):
            return pltpu.make_async_copy(
                table_hbm.at[pl.ds(0, _RCH)], rows_v.at[buf], gsem.at[buf]
            )

        idx_copy(0, 0).start()

        def body(ch, carry):
            buf = lax.rem(ch, 2)
            obuf = 1 - buf

            idx_copy(ch, buf).wait()

            @pl.when(ch >= 2)
            def _():
                out_copy(ch - 2, buf).wait()

            for j in range(_KCH):
                pltpu.make_async_copy(
                    table_hbm.at[idx_v.at[buf, j]],
                    rows_v.at[buf, pl.ds(j * _L, _L)],
                    gsem.at[buf],
                ).start()

            @pl.when(ch == 0)
            def _():
                idx_copy(1, 1).start()

            @pl.when(ch >= 1)
            def _():
                gather_drain(obuf).wait()
                out_copy(ch - 1, obuf).start()

                @pl.when(ch + 1 < n_ch)
                def _():
                    idx_copy(ch + 1, obuf).start()

            return carry

        lax.fori_loop(0, n_ch, body, 0)

        last = n_ch - 1
        lbuf = last % 2
        gather_drain(lbuf).wait()
        out_copy(last, lbuf).start()
        out_copy(last - 1, 1 - lbuf).wait()
        out_copy(last, lbuf).wait()

    return gather_kernel


def kernel(x, pe):
    V, D = pe.shape
    B = x.size
    xf = x.reshape(-1).astype(jnp.int32)
    idx3 = xf.reshape(_NW, B // (_NW * _L), _L)
    out = _build(V, D, B)(idx3, pe)
    return out.reshape(*x.shape, D)
